# Initial kernel scaffold; baseline (speedup 1.0000x reference)
import functools

import jax
import jax.numpy as jnp
from jax import lax
from jax.experimental import pallas as pl
from jax.experimental.pallas import tpu as pltpu
from jax.experimental.pallas import tpu_sc as plsc

NC, NS, LANES = 2, 16, 16
NW = NC * NS
D = 32
N = 4096 * 200
N_PER_W = N // NW
C = 1024
G = 128
CHUNKS = N_PER_W // C
SUB = C // G


def _dual_embed_body(ids_hbm, p_hbm, l_hbm, out_hbm, idx_v, bufp, bufl, sem):
    wid = lax.axis_index("s") * NC + lax.axis_index("c")
    base = wid * N_PER_W

    @pl.loop(0, CHUNKS)
    def _chunk(g):
        off = base + g * C
        pltpu.sync_copy(ids_hbm.at[pl.ds(off, C)], idx_v)
        idx2 = idx_v.reshape(SUB, G)
        copies = []
        for j in range(SUB):
            dst = pl.ds(j * G, G)
            copies.append(pltpu.async_copy(p_hbm.at[idx2.at[j]], bufp.at[dst], sem))
            copies.append(pltpu.async_copy(l_hbm.at[idx2.at[j]], bufl.at[dst], sem))
        for cp in copies:
            cp.wait()

        @pl.loop(0, C, unroll=4)
        def _row(i):
            bufp[i, 0:LANES] += bufl[i, 0:LANES]
            bufp[i, LANES:D] += bufl[i, LANES:D]

        pltpu.sync_copy(bufp, out_hbm.at[pl.ds(off, C)])


@functools.partial(
    pl.kernel,
    out_type=jax.ShapeDtypeStruct((N, D), jnp.float32),
    mesh=plsc.VectorSubcoreMesh(core_axis_name="c", subcore_axis_name="s"),
    scratch_types=[
        pltpu.VMEM((C,), jnp.int32),
        pltpu.VMEM((C, D), jnp.float32),
        pltpu.VMEM((C, D), jnp.float32),
        pltpu.SemaphoreType.DMA,
    ],
)
def _dual_embed(ids_hbm, p_hbm, l_hbm, out_hbm, idx_v, bufp, bufl, sem):
    _dual_embed_body(ids_hbm, p_hbm, l_hbm, out_hbm, idx_v, bufp, bufl, sem)


@jax.jit
def kernel(input_ids, pretrained_weight, learnable_weight):
    ids = input_ids.reshape(-1).astype(jnp.int32)
    out = _dual_embed(ids, pretrained_weight, learnable_weight)
    return out.reshape(input_ids.shape + (D,))


# speedup vs baseline: 1.5417x; 1.5417x over previous
import functools

import jax
import jax.numpy as jnp
from jax import lax
from jax.experimental import pallas as pl
from jax.experimental.pallas import tpu as pltpu
from jax.experimental.pallas import tpu_sc as plsc

NC, NS, LANES = 2, 16, 16
NW = NC * NS
D = 32
N = 4096 * 200
N_PER_W = N // NW
C = 1024
G = 128
CHUNKS = N_PER_W // C
SUB = C // G


def _dual_embed_body(ids_hbm, p_hbm, l_hbm, out_hbm, idx_v, bufp, bufl, sem):
    wid = lax.axis_index("s") * NC + lax.axis_index("c")
    base = wid * N_PER_W

    @pl.loop(0, CHUNKS)
    def _chunk(g):
        off = base + g * C
        pltpu.sync_copy(ids_hbm.at[pl.ds(pl.multiple_of(off // G, 8), SUB)], idx_v)
        copies = []
        for j in range(SUB):
            dst = pl.ds(j * G, G)
            copies.append(pltpu.async_copy(p_hbm.at[idx_v.at[j]], bufp.at[dst], sem))
            copies.append(pltpu.async_copy(l_hbm.at[idx_v.at[j]], bufl.at[dst], sem))
        for cp in copies:
            cp.wait()

        @pl.loop(0, C, unroll=4)
        def _row(i):
            bufp[i, 0:LANES] += bufl[i, 0:LANES]
            bufp[i, LANES:D] += bufl[i, LANES:D]

        pltpu.sync_copy(bufp, out_hbm.at[pl.ds(off, C)])


@functools.partial(
    pl.kernel,
    out_type=jax.ShapeDtypeStruct((N, D), jnp.float32),
    mesh=plsc.VectorSubcoreMesh(core_axis_name="c", subcore_axis_name="s"),
    scratch_types=[
        pltpu.VMEM((SUB, G), jnp.int32),
        pltpu.VMEM((C, D), jnp.float32),
        pltpu.VMEM((C, D), jnp.float32),
        pltpu.SemaphoreType.DMA,
    ],
    compiler_params=pltpu.CompilerParams(use_tc_tiling_on_sc=False),
)
def _dual_embed(ids_hbm, p_hbm, l_hbm, out_hbm, idx_v, bufp, bufl, sem):
    _dual_embed_body(ids_hbm, p_hbm, l_hbm, out_hbm, idx_v, bufp, bufl, sem)


@jax.jit
def kernel(input_ids, pretrained_weight, learnable_weight):
    ids = input_ids.reshape(-1, G).astype(jnp.int32)
    out = _dual_embed(ids, pretrained_weight, learnable_weight)
    return out.reshape(input_ids.shape + (D,))


# speedup vs baseline: 1.7243x; 1.1185x over previous
import functools

import jax
import jax.numpy as jnp
from jax import lax
from jax.experimental import pallas as pl
from jax.experimental.pallas import tpu as pltpu
from jax.experimental.pallas import tpu_sc as plsc

NC, NS, LANES = 2, 16, 16
NW = NC * NS
D = 32
N = 4096 * 200
N_PER_W = N // NW
G = 128
ROWS = N_PER_W // G
C = 512
SUB = C // G
CHUNKS = N_PER_W // C


def _dual_embed_body(ids_hbm, p_hbm, l_hbm, out_hbm,
                     idx_v, bufp, bufl, gsem0, gsem1, ssem0, ssem1):
    gsems = (gsem0, gsem1)
    ssems = (ssem0, ssem1)
    wid = lax.axis_index("s") * NC + lax.axis_index("c")
    base = wid * N_PER_W

    pltpu.sync_copy(ids_hbm.at[pl.ds(pl.multiple_of(base // G, 8), ROWS)], idx_v)

    def issue_gathers(chunk, slot):
        for j in range(SUB):
            row = chunk * SUB + j
            dst = pl.ds(j * G, G)
            pltpu.async_copy(p_hbm.at[idx_v.at[row]], bufp.at[slot].at[dst], gsems[slot])
            pltpu.async_copy(l_hbm.at[idx_v.at[row]], bufl.at[slot].at[dst], gsems[slot])

    def wait_gathers(slot):
        pltpu.make_async_copy(p_hbm.at[pl.ds(0, C)], bufp.at[slot], gsems[slot]).wait()
        pltpu.make_async_copy(l_hbm.at[pl.ds(0, C)], bufl.at[slot], gsems[slot]).wait()

    def add_rows(slot):
        bp = bufp.at[slot]
        bl = bufl.at[slot]

        @pl.loop(0, C, unroll=8)
        def _row(i):
            bp[i, 0:LANES] += bl[i, 0:LANES]
            bp[i, LANES:D] += bl[i, LANES:D]

    def issue_store(chunk, slot):
        off = base + chunk * C
        pltpu.async_copy(bufp.at[slot], out_hbm.at[pl.ds(off, C)], ssems[slot])

    def wait_store(slot):
        pltpu.make_async_copy(p_hbm.at[pl.ds(0, C)], bufp.at[slot], ssems[slot]).wait()

    issue_gathers(0, 0)
    issue_gathers(1, 1)
    wait_gathers(0)
    add_rows(0)
    issue_store(0, 0)

    @pl.loop(1, CHUNKS - 1, step=2)
    def _pair(g):
        for b in range(2):
            chunk = g + b
            slot = (1 + b) % 2
            wait_store(1 - slot)
            issue_gathers(chunk + 1, 1 - slot)
            wait_gathers(slot)
            add_rows(slot)
            issue_store(chunk, slot)

    wait_store(0)
    wait_gathers(1)
    add_rows(1)
    issue_store(CHUNKS - 1, 1)
    wait_store(1)


@functools.partial(
    pl.kernel,
    out_type=jax.ShapeDtypeStruct((N, D), jnp.float32),
    mesh=plsc.VectorSubcoreMesh(core_axis_name="c", subcore_axis_name="s"),
    scratch_types=[
        pltpu.VMEM((ROWS, G), jnp.int32),
        pltpu.VMEM((2, C, D), jnp.float32),
        pltpu.VMEM((2, C, D), jnp.float32),
        pltpu.SemaphoreType.DMA,
        pltpu.SemaphoreType.DMA,
        pltpu.SemaphoreType.DMA,
        pltpu.SemaphoreType.DMA,
    ],
    compiler_params=pltpu.CompilerParams(use_tc_tiling_on_sc=False),
)
def _dual_embed(ids_hbm, p_hbm, l_hbm, out_hbm,
                idx_v, bufp, bufl, gsem0, gsem1, ssem0, ssem1):
    _dual_embed_body(ids_hbm, p_hbm, l_hbm, out_hbm,
                     idx_v, bufp, bufl, gsem0, gsem1, ssem0, ssem1)


@jax.jit
def kernel(input_ids, pretrained_weight, learnable_weight):
    ids = input_ids.reshape(-1, G).astype(jnp.int32)
    out = _dual_embed(ids, pretrained_weight, learnable_weight)
    return out.reshape(input_ids.shape + (D,))
